# baseline (speedup 1.0000x reference)
import functools

import jax
import jax.numpy as jnp
from jax import lax
from jax.experimental import pallas as pl
from jax.experimental.pallas import tpu as pltpu
from jax.experimental.pallas import tpu_sc as plsc

N = 10000
NP = 10240
E = 320000
TILES = 16
CORES = 2
CHUNK = 128
CHUNKS = 160
GROUP = 128
EPT = CHUNKS * CHUNK
NGRP = EPT // GROUP
EPAD = TILES * CHUNKS * CHUNK
DCHUNKS = 80
DPAD = CORES * TILES * DCHUNKS * CHUNK
STRIPE = NP // TILES
ROWS_BLK = 512
NBLK = NP // ROWS_BLK


def _mesh():
    return plsc.VectorSubcoreMesh(core_axis_name="c", subcore_axis_name="s")


def _deg_body(dst_hbm, deg_out, idx_d, ones_v, zbuf, acc):
    c = lax.axis_index("c")
    s = lax.axis_index("s")

    def zb(i, carry):
        zbuf[pl.ds(i * 16, 16)] = jnp.zeros((16,), jnp.float32)
        return carry

    lax.fori_loop(0, STRIPE // 16, zb, 0)
    for i in range(GROUP // 16):
        ones_v[pl.ds(i * 16, 16)] = jnp.ones((16,), jnp.float32)
    pltpu.sync_copy(zbuf, acc.at[pl.ds(s * STRIPE, STRIPE)])
    pltpu.sync_copy(dst_hbm.at[s], idx_d)
    plsc.subcore_barrier()

    def body(j, carry):
        pltpu.sync_copy(ones_v, acc.at[idx_d.at[j]], add=True)
        return carry

    lax.fori_loop(c * (NGRP // 2), (c + 1) * (NGRP // 2), body, 0)
    plsc.subcore_barrier()
    pltpu.sync_copy(acc.at[pl.ds(s * STRIPE, STRIPE)],
                    deg_out.at[c, pl.ds(s * STRIPE, STRIPE)])


def _make_deg():
    return pl.kernel(
        _deg_body,
        out_type=jax.ShapeDtypeStruct((CORES, NP), jnp.float32),
        mesh=_mesh(),
        scratch_types=[
            pltpu.VMEM((NGRP, GROUP), jnp.int32),
            pltpu.VMEM((GROUP,), jnp.float32),
            pltpu.VMEM((STRIPE,), jnp.float32),
            pltpu.VMEM_SHARED((NP,), jnp.float32),
        ],
        compiler_params=pltpu.CompilerParams(use_tc_tiling_on_sc=False),
    )


def _agg_body(fh, passes, y_hbm, src_hbm, dst_hbm, agg_out,
              idx_s, idx_d, rows0, rows1, zbuf, acc, sem0, sem1):
    c = lax.axis_index("c")
    s = lax.axis_index("s")
    zr = zbuf.shape[0]

    def zb(i, carry):
        for k in range(fh // 16):
            zbuf[i, pl.ds(k * 16, 16)] = jnp.zeros((16,), jnp.float32)
        return carry

    lax.fori_loop(0, zr, zb, 0)
    pltpu.sync_copy(src_hbm.at[s], idx_s)
    pltpu.sync_copy(dst_hbm.at[s], idx_d)

    def _shift(off):
        def shift(i, carry):
            for k in range(GROUP // 16):
                idx_s[i, pl.ds(k * 16, 16)] = (
                    idx_s[i, pl.ds(k * 16, 16)] + off)
            return carry

        lax.fori_loop(0, NGRP, shift, 0)

    _shift(c * NP)

    def _gat(j, buf, sem):
        pltpu.async_copy(y_hbm.at[idx_s.at[j]], buf, sem)

    def _gwait(j, buf, sem):
        pltpu.make_async_copy(y_hbm.at[idx_s.at[j]], buf, sem).wait()

    def _sca(j, buf):
        pltpu.sync_copy(buf, acc.at[idx_d.at[j]], add=True)

    for p in range(passes):
        if p > 0:
            _shift(CORES * NP)

        def zs(j, carry):
            pltpu.sync_copy(zbuf, acc.at[pl.ds(s * STRIPE + j * zr, zr)])
            return carry

        lax.fori_loop(0, STRIPE // zr, zs, 0)
        plsc.subcore_barrier()

        _gat(0, rows0, sem0)

        def body(jj, carry):
            j0 = jj * 2
            _gat(j0 + 1, rows1, sem1)
            _gwait(j0, rows0, sem0)
            _sca(j0, rows0)

            @pl.when(jj < NGRP // 2 - 1)
            def _():
                _gat(j0 + 2, rows0, sem0)

            _gwait(j0 + 1, rows1, sem1)
            _sca(j0 + 1, rows1)
            return carry

        lax.fori_loop(0, NGRP // 2, body, 0)
        plsc.subcore_barrier()
        pltpu.sync_copy(acc.at[pl.ds(s * STRIPE, STRIPE)],
                        agg_out.at[CORES * p + c, pl.ds(s * STRIPE, STRIPE)])


def _make_agg(fh, passes):
    return pl.kernel(
        functools.partial(_agg_body, fh, passes),
        out_type=jax.ShapeDtypeStruct((CORES * passes, NP, fh), jnp.float32),
        mesh=_mesh(),
        scratch_types=[
            pltpu.VMEM((NGRP, GROUP), jnp.int32),
            pltpu.VMEM((NGRP, GROUP), jnp.int32),
            pltpu.VMEM((GROUP, fh), jnp.float32),
            pltpu.VMEM((GROUP, fh), jnp.float32),
            pltpu.VMEM((64, fh), jnp.float32),
            pltpu.VMEM_SHARED((NP, fh), jnp.float32),
            pltpu.SemaphoreType.DMA,
            pltpu.SemaphoreType.DMA,
        ],
        compiler_params=pltpu.CompilerParams(use_tc_tiling_on_sc=False),
    )


_DEG = _make_deg()
_AGG64 = _make_agg(64, 2)
_AGG32 = _make_agg(32, 1)


def _split_store(o_ref, y):
    ns = o_ref.shape[0]
    fh = y.shape[1] // ns
    for q in range(ns):
        o_ref[q] = y[:, q * fh:(q + 1) * fh]


def _merge(agg_ref, y_ref, dis):
    ns = agg_ref.shape[0]
    return jnp.concatenate(
        [(agg_ref[q] + y_ref[q]) * dis for q in range(ns)], axis=1)


def _mm1_body(x_ref, w_ref, d0_ref, d1_ref, y_ref):
    dis = lax.rsqrt(d0_ref[...] + d1_ref[...] + 1.0)
    y = jnp.dot(x_ref[...], w_ref[...], preferred_element_type=jnp.float32) * dis
    _split_store(y_ref, y)


def _mid_body(agg_ref, y_ref, d0_ref, d1_ref, b_ref, w_ref, o_ref):
    dis = lax.rsqrt(d0_ref[...] + d1_ref[...] + 1.0)
    h = jnp.maximum(_merge(agg_ref, y_ref, dis) + b_ref[...], 0.0)
    y = jnp.dot(h, w_ref[...], preferred_element_type=jnp.float32) * dis
    _split_store(o_ref, y)


def _fin_body(agg_ref, y_ref, d0_ref, d1_ref, b_ref, o_ref):
    dis = lax.rsqrt(d0_ref[...] + d1_ref[...] + 1.0)
    z = _merge(agg_ref, y_ref, dis) + b_ref[...]
    z = z - jnp.max(z, axis=1, keepdims=True)
    e = jnp.exp(z)
    o_ref[...] = e / jnp.sum(e, axis=1, keepdims=True)


def _col_spec():
    return pl.BlockSpec((ROWS_BLK, 1), lambda i: (i, 0))


def _split_spec(ns, fh):
    return pl.BlockSpec((ns, ROWS_BLK, fh), lambda i: (0, i, 0))


def _mm1(xp, w1, d0, d1, osplit):
    fin, fout = w1.shape
    return pl.pallas_call(
        _mm1_body,
        grid=(NBLK,),
        in_specs=[
            pl.BlockSpec((ROWS_BLK, fin), lambda i: (i, 0)),
            pl.BlockSpec((fin, fout), lambda i: (0, 0)),
            _col_spec(), _col_spec(),
        ],
        out_specs=_split_spec(osplit, fout // osplit),
        out_shape=jax.ShapeDtypeStruct((osplit, NP, fout // osplit), jnp.float32),
    )(xp, w1, d0, d1)


def _mid(agg, y, d0, d1, b, w, osplit):
    fin, fout = w.shape
    ins = agg.shape[0]
    return pl.pallas_call(
        _mid_body,
        grid=(NBLK,),
        in_specs=[
            _split_spec(ins, fin // ins),
            _split_spec(ins, fin // ins),
            _col_spec(), _col_spec(),
            pl.BlockSpec((1, fin), lambda i: (0, 0)),
            pl.BlockSpec((fin, fout), lambda i: (0, 0)),
        ],
        out_specs=_split_spec(osplit, fout // osplit),
        out_shape=jax.ShapeDtypeStruct((osplit, NP, fout // osplit), jnp.float32),
    )(agg, y, d0, d1, b, w)


def _fin(agg, y, d0, d1, b):
    fout = b.shape[1]
    ins = agg.shape[0]
    return pl.pallas_call(
        _fin_body,
        grid=(NBLK,),
        in_specs=[
            _split_spec(ins, fout // ins),
            _split_spec(ins, fout // ins),
            _col_spec(), _col_spec(),
            pl.BlockSpec((1, fout), lambda i: (0, 0)),
        ],
        out_specs=pl.BlockSpec((ROWS_BLK, fout), lambda i: (i, 0)),
        out_shape=jax.ShapeDtypeStruct((NP, fout), jnp.float32),
    )(agg, y, d0, d1, b)


def kernel(x, edge_index, W1, b1, W2, b2, W3, b3):
    src = edge_index[0].astype(jnp.int32)
    dst = edge_index[1].astype(jnp.int32)
    npad = EPAD - E
    fill = jnp.arange(npad, dtype=jnp.int32) % 8
    src_t = jnp.concatenate([src, fill]).reshape(TILES, NGRP, GROUP)
    dst_t = jnp.concatenate([dst, N + fill]).reshape(TILES, NGRP, GROUP)

    xp = jnp.pad(x, ((0, NP - N), (0, 0)))

    deg = _DEG(dst_t)
    d0 = deg[0].reshape(NP, 1)
    d1 = deg[1].reshape(NP, 1)

    y1 = _mm1(xp, W1, d0, d1, 4)
    agg1 = _AGG64(y1.reshape(4 * NP, 64), src_t, dst_t)
    y2 = _mid(agg1, y1, d0, d1, b1.reshape(1, -1), W2, 4)
    agg2 = _AGG64(y2.reshape(4 * NP, 64), src_t, dst_t)
    y3 = _mid(agg2, y2, d0, d1, b2.reshape(1, -1), W3, 2)
    agg3 = _AGG32(y3.reshape(2 * NP, 32), src_t, dst_t)
    out = _fin(agg3, y3, d0, d1, b3.reshape(1, -1))
    return out[:N]


# speedup vs baseline: 1.1815x; 1.1815x over previous
import functools

import jax
import jax.numpy as jnp
from jax import lax
from jax.experimental import pallas as pl
from jax.experimental.pallas import tpu as pltpu
from jax.experimental.pallas import tpu_sc as plsc

N = 10000
NP = 10240
E = 320000
TILES = 16
CORES = 2
CHUNK = 128
CHUNKS = 158
EPAD = TILES * CHUNKS * CHUNK
DCHUNKS = 80
DPAD = CORES * TILES * DCHUNKS * CHUNK
STRIPE = NP // TILES
ROWS_BLK = 512
NBLK = NP // ROWS_BLK


def _mesh():
    return plsc.VectorSubcoreMesh(core_axis_name="c", subcore_axis_name="s")


def _deg_body(dst_hbm, deg_out, idx_d, ones_v, zbuf, acc):
    c = lax.axis_index("c")
    s = lax.axis_index("s")

    def zb(i, carry):
        zbuf[pl.ds(i * 16, 16)] = jnp.zeros((16,), jnp.float32)
        return carry

    lax.fori_loop(0, STRIPE // 16, zb, 0)
    for i in range(CHUNK // 16):
        ones_v[pl.ds(i * 16, 16)] = jnp.ones((16,), jnp.float32)
    pltpu.sync_copy(zbuf, acc.at[pl.ds(s * STRIPE, STRIPE)])
    pltpu.sync_copy(dst_hbm.at[c, s], idx_d)
    plsc.subcore_barrier()

    def body(j, carry):
        pltpu.sync_copy(ones_v, acc.at[idx_d.at[j]], add=True)
        return carry

    lax.fori_loop(0, DCHUNKS, body, 0)
    plsc.subcore_barrier()
    pltpu.sync_copy(acc.at[pl.ds(s * STRIPE, STRIPE)],
                    deg_out.at[c, pl.ds(s * STRIPE, STRIPE)])


def _make_deg():
    return pl.kernel(
        _deg_body,
        out_type=jax.ShapeDtypeStruct((CORES, NP), jnp.float32),
        mesh=_mesh(),
        scratch_types=[
            pltpu.VMEM((DCHUNKS, CHUNK), jnp.int32),
            pltpu.VMEM((CHUNK,), jnp.float32),
            pltpu.VMEM((STRIPE,), jnp.float32),
            pltpu.VMEM_SHARED((NP,), jnp.float32),
        ],
        compiler_params=pltpu.CompilerParams(use_tc_tiling_on_sc=False),
    )


def _agg_body(fh, passes, y_hbm, srcoff_hbm, dst_hbm, agg_out,
              idx_s, idx_d, rows0, rows1, zbuf, acc, sem0, sem1):
    c = lax.axis_index("c")
    s = lax.axis_index("s")
    zr = zbuf.shape[0]

    def zb(i, carry):
        for k in range(fh // 16):
            zbuf[i, pl.ds(k * 16, 16)] = jnp.zeros((16,), jnp.float32)
        return carry

    lax.fori_loop(0, zr, zb, 0)
    pltpu.sync_copy(srcoff_hbm.at[c, s], idx_s)
    pltpu.sync_copy(dst_hbm.at[s], idx_d)

    for p in range(passes):
        if p > 0:
            def shift(i, carry):
                for k in range(CHUNK // 16):
                    idx_s[i, pl.ds(k * 16, 16)] = (
                        idx_s[i, pl.ds(k * 16, 16)] + CORES * NP)
                return carry

            lax.fori_loop(0, CHUNKS, shift, 0)

        pltpu.async_copy(y_hbm.at[idx_s.at[0]], rows0, sem0)

        def zs(j, carry):
            pltpu.sync_copy(zbuf, acc.at[pl.ds(s * STRIPE + j * zr, zr)])
            return carry

        lax.fori_loop(0, STRIPE // zr, zs, 0)
        plsc.subcore_barrier()

        def body(jj, carry):
            j0 = jj * 2
            pltpu.async_copy(y_hbm.at[idx_s.at[j0 + 1]], rows1, sem1)
            pltpu.make_async_copy(y_hbm.at[idx_s.at[j0]], rows0, sem0).wait()
            pltpu.sync_copy(rows0, acc.at[idx_d.at[j0]], add=True)

            @pl.when(jj < CHUNKS // 2 - 1)
            def _():
                pltpu.async_copy(y_hbm.at[idx_s.at[j0 + 2]], rows0, sem0)

            pltpu.make_async_copy(y_hbm.at[idx_s.at[j0 + 1]], rows1, sem1).wait()
            pltpu.sync_copy(rows1, acc.at[idx_d.at[j0 + 1]], add=True)
            return carry

        lax.fori_loop(0, CHUNKS // 2, body, 0)
        plsc.subcore_barrier()
        pltpu.sync_copy(acc.at[pl.ds(s * STRIPE, STRIPE)],
                        agg_out.at[CORES * p + c, pl.ds(s * STRIPE, STRIPE)])


def _make_agg(fh, passes):
    return pl.kernel(
        functools.partial(_agg_body, fh, passes),
        out_type=jax.ShapeDtypeStruct((CORES * passes, NP, fh), jnp.float32),
        mesh=_mesh(),
        scratch_types=[
            pltpu.VMEM((CHUNKS, CHUNK), jnp.int32),
            pltpu.VMEM((CHUNKS, CHUNK), jnp.int32),
            pltpu.VMEM((CHUNK, fh), jnp.float32),
            pltpu.VMEM((CHUNK, fh), jnp.float32),
            pltpu.VMEM((64, fh), jnp.float32),
            pltpu.VMEM_SHARED((NP, fh), jnp.float32),
            pltpu.SemaphoreType.DMA,
            pltpu.SemaphoreType.DMA,
        ],
        compiler_params=pltpu.CompilerParams(use_tc_tiling_on_sc=False),
    )


_DEG = _make_deg()
_AGG64 = _make_agg(64, 2)
_AGG32 = _make_agg(32, 1)


def _split_store(o_ref, y):
    ns = o_ref.shape[0]
    fh = y.shape[1] // ns
    for q in range(ns):
        o_ref[q] = y[:, q * fh:(q + 1) * fh]


def _merge(agg_ref, y_ref, dis):
    ns = agg_ref.shape[0]
    return jnp.concatenate(
        [(agg_ref[q] + y_ref[q]) * dis for q in range(ns)], axis=1)


def _mm1_body(x_ref, w_ref, d0_ref, d1_ref, y_ref):
    dis = lax.rsqrt(d0_ref[...] + d1_ref[...] + 1.0)
    y = jnp.dot(x_ref[...], w_ref[...], preferred_element_type=jnp.float32) * dis
    _split_store(y_ref, y)


def _mid_body(agg_ref, y_ref, d0_ref, d1_ref, b_ref, w_ref, o_ref):
    dis = lax.rsqrt(d0_ref[...] + d1_ref[...] + 1.0)
    h = jnp.maximum(_merge(agg_ref, y_ref, dis) + b_ref[...], 0.0)
    y = jnp.dot(h, w_ref[...], preferred_element_type=jnp.float32) * dis
    _split_store(o_ref, y)


def _fin_body(agg_ref, y_ref, d0_ref, d1_ref, b_ref, o_ref):
    dis = lax.rsqrt(d0_ref[...] + d1_ref[...] + 1.0)
    z = _merge(agg_ref, y_ref, dis) + b_ref[...]
    z = z - jnp.max(z, axis=1, keepdims=True)
    e = jnp.exp(z)
    o_ref[...] = e / jnp.sum(e, axis=1, keepdims=True)


def _col_spec():
    return pl.BlockSpec((ROWS_BLK, 1), lambda i: (i, 0))


def _split_spec(ns, fh):
    return pl.BlockSpec((ns, ROWS_BLK, fh), lambda i: (0, i, 0))


def _mm1(xp, w1, d0, d1, osplit):
    fin, fout = w1.shape
    return pl.pallas_call(
        _mm1_body,
        grid=(NBLK,),
        in_specs=[
            pl.BlockSpec((ROWS_BLK, fin), lambda i: (i, 0)),
            pl.BlockSpec((fin, fout), lambda i: (0, 0)),
            _col_spec(), _col_spec(),
        ],
        out_specs=_split_spec(osplit, fout // osplit),
        out_shape=jax.ShapeDtypeStruct((osplit, NP, fout // osplit), jnp.float32),
    )(xp, w1, d0, d1)


def _mid(agg, y, d0, d1, b, w, osplit):
    fin, fout = w.shape
    ins = agg.shape[0]
    return pl.pallas_call(
        _mid_body,
        grid=(NBLK,),
        in_specs=[
            _split_spec(ins, fin // ins),
            _split_spec(ins, fin // ins),
            _col_spec(), _col_spec(),
            pl.BlockSpec((1, fin), lambda i: (0, 0)),
            pl.BlockSpec((fin, fout), lambda i: (0, 0)),
        ],
        out_specs=_split_spec(osplit, fout // osplit),
        out_shape=jax.ShapeDtypeStruct((osplit, NP, fout // osplit), jnp.float32),
    )(agg, y, d0, d1, b, w)


def _fin(agg, y, d0, d1, b):
    fout = b.shape[1]
    ins = agg.shape[0]
    return pl.pallas_call(
        _fin_body,
        grid=(NBLK,),
        in_specs=[
            _split_spec(ins, fout // ins),
            _split_spec(ins, fout // ins),
            _col_spec(), _col_spec(),
            pl.BlockSpec((1, fout), lambda i: (0, 0)),
        ],
        out_specs=pl.BlockSpec((ROWS_BLK, fout), lambda i: (i, 0)),
        out_shape=jax.ShapeDtypeStruct((NP, fout), jnp.float32),
    )(agg, y, d0, d1, b)


def kernel(x, edge_index, W1, b1, W2, b2, W3, b3):
    src = edge_index[0].astype(jnp.int32)
    dst = edge_index[1].astype(jnp.int32)
    npad = EPAD - E
    fill = jnp.arange(npad, dtype=jnp.int32) % 8
    src_t = jnp.concatenate([src, fill]).reshape(TILES, CHUNKS, CHUNK)
    dst_t = jnp.concatenate([dst, N + fill]).reshape(TILES, CHUNKS, CHUNK)
    srcoff = jnp.stack([src_t, src_t + NP])
    dfill = jnp.arange(DPAD - E, dtype=jnp.int32) % 8
    dst_d = jnp.concatenate([dst, N + dfill]).reshape(CORES, TILES, DCHUNKS, CHUNK)

    xp = jnp.pad(x, ((0, NP - N), (0, 0)))

    deg = _DEG(dst_d)
    d0 = deg[0].reshape(NP, 1)
    d1 = deg[1].reshape(NP, 1)

    y1 = _mm1(xp, W1, d0, d1, 4)
    agg1 = _AGG64(y1.reshape(4 * NP, 64), srcoff, dst_t)
    y2 = _mid(agg1, y1, d0, d1, b1.reshape(1, -1), W2, 4)
    agg2 = _AGG64(y2.reshape(4 * NP, 64), srcoff, dst_t)
    y3 = _mid(agg2, y2, d0, d1, b2.reshape(1, -1), W3, 2)
    agg3 = _AGG32(y3.reshape(2 * NP, 32), srcoff, dst_t)
    out = _fin(agg3, y3, d0, d1, b3.reshape(1, -1))
    return out[:N]


# speedup vs baseline: 1.1832x; 1.0014x over previous
import functools

import jax
import jax.numpy as jnp
from jax import lax
from jax.experimental import pallas as pl
from jax.experimental.pallas import tpu as pltpu
from jax.experimental.pallas import tpu_sc as plsc

N = 10000
NP = 10240
E = 320000
TILES = 16
CORES = 2
CHUNK = 128
CHUNKS = 159
EPAD = TILES * CHUNKS * CHUNK
DCHUNKS = 80
DPAD = CORES * TILES * DCHUNKS * CHUNK
STRIPE = NP // TILES
ROWS_BLK = 512
NBLK = NP // ROWS_BLK


def _mesh():
    return plsc.VectorSubcoreMesh(core_axis_name="c", subcore_axis_name="s")


def _deg_body(dst_hbm, deg_out, idx_d, ones_v, zbuf, acc):
    c = lax.axis_index("c")
    s = lax.axis_index("s")

    def zb(i, carry):
        zbuf[pl.ds(i * 16, 16)] = jnp.zeros((16,), jnp.float32)
        return carry

    lax.fori_loop(0, STRIPE // 16, zb, 0)
    for i in range(CHUNK // 16):
        ones_v[pl.ds(i * 16, 16)] = jnp.ones((16,), jnp.float32)
    pltpu.sync_copy(zbuf, acc.at[pl.ds(s * STRIPE, STRIPE)])
    pltpu.sync_copy(dst_hbm.at[c, s], idx_d)
    plsc.subcore_barrier()

    def body(j, carry):
        pltpu.sync_copy(ones_v, acc.at[idx_d.at[j]], add=True)
        return carry

    lax.fori_loop(0, DCHUNKS, body, 0)
    plsc.subcore_barrier()
    pltpu.sync_copy(acc.at[pl.ds(s * STRIPE, STRIPE)],
                    deg_out.at[c, pl.ds(s * STRIPE, STRIPE)])


def _make_deg():
    return pl.kernel(
        _deg_body,
        out_type=jax.ShapeDtypeStruct((CORES, NP), jnp.float32),
        mesh=_mesh(),
        scratch_types=[
            pltpu.VMEM((DCHUNKS, CHUNK), jnp.int32),
            pltpu.VMEM((CHUNK,), jnp.float32),
            pltpu.VMEM((STRIPE,), jnp.float32),
            pltpu.VMEM_SHARED((NP,), jnp.float32),
        ],
        compiler_params=pltpu.CompilerParams(use_tc_tiling_on_sc=False),
    )


def _agg_body(fh, passes, y_hbm, srcoff_hbm, dst_hbm, agg_out,
              idx_s, idx_d, r0, r1, r2, zbuf, acc,
              g0, g1, g2, s0, s1, s2):
    rows = [r0, r1, r2]
    gsem = [g0, g1, g2]
    ssem = [s0, s1, s2]
    c = lax.axis_index("c")
    s = lax.axis_index("s")
    zr = zbuf.shape[0]
    ngrp = CHUNKS // 3

    def zb(i, carry):
        for k in range(fh // 16):
            zbuf[i, pl.ds(k * 16, 16)] = jnp.zeros((16,), jnp.float32)
        return carry

    lax.fori_loop(0, zr, zb, 0)
    pltpu.sync_copy(srcoff_hbm.at[c, s], idx_s)
    pltpu.sync_copy(dst_hbm.at[s], idx_d)

    def _gat(j, b):
        pltpu.async_copy(y_hbm.at[idx_s.at[j]], rows[b], gsem[b])

    def _gwait(j, b):
        pltpu.make_async_copy(y_hbm.at[idx_s.at[j]], rows[b], gsem[b]).wait()

    def _sstart(j, b):
        pltpu.async_copy(rows[b], acc.at[idx_d.at[j]], ssem[b], add=True)

    def _swait(j, b):
        pltpu.make_async_copy(rows[b], acc.at[idx_d.at[j]], ssem[b]).wait()

    for p in range(passes):
        if p > 0:
            def shift(i, carry):
                for k in range(CHUNK // 16):
                    idx_s[i, pl.ds(k * 16, 16)] = (
                        idx_s[i, pl.ds(k * 16, 16)] + CORES * NP)
                return carry

            lax.fori_loop(0, CHUNKS, shift, 0)

        _gat(0, 0)
        _gat(1, 1)

        def zs(j, carry):
            pltpu.sync_copy(zbuf, acc.at[pl.ds(s * STRIPE + j * zr, zr)])
            return carry

        lax.fori_loop(0, STRIPE // zr, zs, 0)
        plsc.subcore_barrier()

        def body(g, carry):
            for t in range(3):
                j = g * 3 + t
                _gwait(j, t)
                _sstart(j, t)
                bn = (t + 2) % 3
                if t == 0:
                    @pl.when(g > 0)
                    def _():
                        _swait(g * 3 - 1, bn)

                    _gat(j + 2, bn)
                else:
                    @pl.when(g < ngrp - 1)
                    def _():
                        _swait(j - 1, bn)
                        _gat(j + 2, bn)
            return carry

        lax.fori_loop(0, ngrp, body, 0)
        _swait(CHUNKS - 3, 0)
        _swait(CHUNKS - 2, 1)
        _swait(CHUNKS - 1, 2)
        plsc.subcore_barrier()
        pltpu.sync_copy(acc.at[pl.ds(s * STRIPE, STRIPE)],
                        agg_out.at[CORES * p + c, pl.ds(s * STRIPE, STRIPE)])


def _make_agg(fh, passes):
    return pl.kernel(
        functools.partial(_agg_body, fh, passes),
        out_type=jax.ShapeDtypeStruct((CORES * passes, NP, fh), jnp.float32),
        mesh=_mesh(),
        scratch_types=[
            pltpu.VMEM((CHUNKS, CHUNK), jnp.int32),
            pltpu.VMEM((CHUNKS, CHUNK), jnp.int32),
            pltpu.VMEM((CHUNK, fh), jnp.float32),
            pltpu.VMEM((CHUNK, fh), jnp.float32),
            pltpu.VMEM((CHUNK, fh), jnp.float32),
            pltpu.VMEM((64, fh), jnp.float32),
            pltpu.VMEM_SHARED((NP, fh), jnp.float32),
            pltpu.SemaphoreType.DMA,
            pltpu.SemaphoreType.DMA,
            pltpu.SemaphoreType.DMA,
            pltpu.SemaphoreType.DMA,
            pltpu.SemaphoreType.DMA,
            pltpu.SemaphoreType.DMA,
        ],
        compiler_params=pltpu.CompilerParams(use_tc_tiling_on_sc=False),
    )


_DEG = _make_deg()
_AGG64 = _make_agg(64, 2)
_AGG32 = _make_agg(32, 1)


def _split_store(o_ref, y):
    ns = o_ref.shape[0]
    fh = y.shape[1] // ns
    for q in range(ns):
        o_ref[q] = y[:, q * fh:(q + 1) * fh]


def _merge(agg_ref, y_ref, dis):
    ns = agg_ref.shape[0]
    return jnp.concatenate(
        [(agg_ref[q] + y_ref[q]) * dis for q in range(ns)], axis=1)


def _mm1_body(x_ref, w_ref, d0_ref, d1_ref, y_ref):
    dis = lax.rsqrt(d0_ref[...] + d1_ref[...] + 1.0)
    y = jnp.dot(x_ref[...], w_ref[...], preferred_element_type=jnp.float32) * dis
    _split_store(y_ref, y)


def _mid_body(agg_ref, y_ref, d0_ref, d1_ref, b_ref, w_ref, o_ref):
    dis = lax.rsqrt(d0_ref[...] + d1_ref[...] + 1.0)
    h = jnp.maximum(_merge(agg_ref, y_ref, dis) + b_ref[...], 0.0)
    y = jnp.dot(h, w_ref[...], preferred_element_type=jnp.float32) * dis
    _split_store(o_ref, y)


def _fin_body(agg_ref, y_ref, d0_ref, d1_ref, b_ref, o_ref):
    dis = lax.rsqrt(d0_ref[...] + d1_ref[...] + 1.0)
    z = _merge(agg_ref, y_ref, dis) + b_ref[...]
    z = z - jnp.max(z, axis=1, keepdims=True)
    e = jnp.exp(z)
    o_ref[...] = e / jnp.sum(e, axis=1, keepdims=True)


def _col_spec():
    return pl.BlockSpec((ROWS_BLK, 1), lambda i: (i, 0))


def _split_spec(ns, fh):
    return pl.BlockSpec((ns, ROWS_BLK, fh), lambda i: (0, i, 0))


def _mm1(xp, w1, d0, d1, osplit):
    fin, fout = w1.shape
    return pl.pallas_call(
        _mm1_body,
        grid=(NBLK,),
        in_specs=[
            pl.BlockSpec((ROWS_BLK, fin), lambda i: (i, 0)),
            pl.BlockSpec((fin, fout), lambda i: (0, 0)),
            _col_spec(), _col_spec(),
        ],
        out_specs=_split_spec(osplit, fout // osplit),
        out_shape=jax.ShapeDtypeStruct((osplit, NP, fout // osplit), jnp.float32),
    )(xp, w1, d0, d1)


def _mid(agg, y, d0, d1, b, w, osplit):
    fin, fout = w.shape
    ins = agg.shape[0]
    return pl.pallas_call(
        _mid_body,
        grid=(NBLK,),
        in_specs=[
            _split_spec(ins, fin // ins),
            _split_spec(ins, fin // ins),
            _col_spec(), _col_spec(),
            pl.BlockSpec((1, fin), lambda i: (0, 0)),
            pl.BlockSpec((fin, fout), lambda i: (0, 0)),
        ],
        out_specs=_split_spec(osplit, fout // osplit),
        out_shape=jax.ShapeDtypeStruct((osplit, NP, fout // osplit), jnp.float32),
    )(agg, y, d0, d1, b, w)


def _fin(agg, y, d0, d1, b):
    fout = b.shape[1]
    ins = agg.shape[0]
    return pl.pallas_call(
        _fin_body,
        grid=(NBLK,),
        in_specs=[
            _split_spec(ins, fout // ins),
            _split_spec(ins, fout // ins),
            _col_spec(), _col_spec(),
            pl.BlockSpec((1, fout), lambda i: (0, 0)),
        ],
        out_specs=pl.BlockSpec((ROWS_BLK, fout), lambda i: (i, 0)),
        out_shape=jax.ShapeDtypeStruct((NP, fout), jnp.float32),
    )(agg, y, d0, d1, b)


def kernel(x, edge_index, W1, b1, W2, b2, W3, b3):
    src = edge_index[0].astype(jnp.int32)
    dst = edge_index[1].astype(jnp.int32)
    npad = EPAD - E
    fill = jnp.arange(npad, dtype=jnp.int32) % 8
    src_t = jnp.concatenate([src, fill]).reshape(TILES, CHUNKS, CHUNK)
    dst_t = jnp.concatenate([dst, N + fill]).reshape(TILES, CHUNKS, CHUNK)
    srcoff = jnp.stack([src_t, src_t + NP])
    dfill = jnp.arange(DPAD - E, dtype=jnp.int32) % 8
    dst_d = jnp.concatenate([dst, N + dfill]).reshape(CORES, TILES, DCHUNKS, CHUNK)

    xp = jnp.pad(x, ((0, NP - N), (0, 0)))

    deg = _DEG(dst_d)
    d0 = deg[0].reshape(NP, 1)
    d1 = deg[1].reshape(NP, 1)

    y1 = _mm1(xp, W1, d0, d1, 4)
    agg1 = _AGG64(y1.reshape(4 * NP, 64), srcoff, dst_t)
    y2 = _mid(agg1, y1, d0, d1, b1.reshape(1, -1), W2, 4)
    agg2 = _AGG64(y2.reshape(4 * NP, 64), srcoff, dst_t)
    y3 = _mid(agg2, y2, d0, d1, b2.reshape(1, -1), W3, 2)
    agg3 = _AGG32(y3.reshape(2 * NP, 32), srcoff, dst_t)
    out = _fin(agg3, y3, d0, d1, b3.reshape(1, -1))
    return out[:N]


# speedup vs baseline: 1.4345x; 1.2124x over previous
import functools

import jax
import jax.numpy as jnp
from jax import lax
from jax.experimental import pallas as pl
from jax.experimental.pallas import tpu as pltpu
from jax.experimental.pallas import tpu_sc as plsc

N = 10000
NP = 10240
E = 320000
TILES = 16
CORES = 2
CHUNK = 128
CHUNKS = 159
EPAD = TILES * CHUNKS * CHUNK
DCHUNKS = 80
DPAD = CORES * TILES * DCHUNKS * CHUNK
STRIPE = NP // TILES
ROWS_BLK = 512
NBLK = NP // ROWS_BLK


def _mesh():
    return plsc.VectorSubcoreMesh(core_axis_name="c", subcore_axis_name="s")


def _deg_body(dst_hbm, deg_out, idx_d, ones_v, zbuf, acc):
    c = lax.axis_index("c")
    s = lax.axis_index("s")

    def zb(i, carry):
        zbuf[pl.ds(i * 16, 16)] = jnp.zeros((16,), jnp.float32)
        return carry

    lax.fori_loop(0, STRIPE // 16, zb, 0)
    for i in range(CHUNK // 16):
        ones_v[pl.ds(i * 16, 16)] = jnp.ones((16,), jnp.float32)
    pltpu.sync_copy(zbuf, acc.at[pl.ds(s * STRIPE, STRIPE)])
    pltpu.sync_copy(dst_hbm.at[c, s], idx_d)
    plsc.subcore_barrier()

    def body(j, carry):
        pltpu.sync_copy(ones_v, acc.at[idx_d.at[j]], add=True)
        return carry

    lax.fori_loop(0, DCHUNKS, body, 0)
    plsc.subcore_barrier()
    pltpu.sync_copy(acc.at[pl.ds(s * STRIPE, STRIPE)],
                    deg_out.at[c, pl.ds(s * STRIPE, STRIPE)])


def _make_deg():
    return pl.kernel(
        _deg_body,
        out_type=jax.ShapeDtypeStruct((CORES, NP), jnp.float32),
        mesh=_mesh(),
        scratch_types=[
            pltpu.VMEM((DCHUNKS, CHUNK), jnp.int32),
            pltpu.VMEM((CHUNK,), jnp.float32),
            pltpu.VMEM((STRIPE,), jnp.float32),
            pltpu.VMEM_SHARED((NP,), jnp.float32),
        ],
        compiler_params=pltpu.CompilerParams(use_tc_tiling_on_sc=False),
    )


def _agg_body(fh, passes, y_hbm, srcoff_hbm, dst_hbm, agg_out,
              idx_s, idx_d, r0, r1, r2, zbuf, acc,
              g0, g1, g2, s0, s1, s2):
    rows = [r0, r1, r2]
    gsem = [g0, g1, g2]
    ssem = [s0, s1, s2]
    c = lax.axis_index("c")
    s = lax.axis_index("s")
    zr = zbuf.shape[0]
    ngrp = CHUNKS // 3

    def zb(i, carry):
        for k in range(fh // 16):
            zbuf[i, pl.ds(k * 16, 16)] = jnp.zeros((16,), jnp.float32)
        return carry

    lax.fori_loop(0, zr, zb, 0)
    pltpu.sync_copy(srcoff_hbm.at[c, s], idx_s)
    pltpu.sync_copy(dst_hbm.at[s], idx_d)

    def _gat(j, b):
        pltpu.async_copy(y_hbm.at[idx_s.at[j]], rows[b], gsem[b])

    def _gwait(j, b):
        pltpu.make_async_copy(y_hbm.at[idx_s.at[j]], rows[b], gsem[b]).wait()

    def _sstart(j, b):
        pltpu.async_copy(rows[b], acc.at[idx_d.at[j]], ssem[b], add=True)

    def _swait(j, b):
        pltpu.make_async_copy(rows[b], acc.at[idx_d.at[j]], ssem[b]).wait()

    for p in range(passes):
        if p > 0:
            def shift(i, carry):
                for k in range(CHUNK // 16):
                    idx_s[i, pl.ds(k * 16, 16)] = (
                        idx_s[i, pl.ds(k * 16, 16)] + CORES * NP)
                return carry

            lax.fori_loop(0, CHUNKS, shift, 0)

        _gat(0, 0)
        _gat(1, 1)

        def zs(j, carry):
            pltpu.sync_copy(zbuf, acc.at[pl.ds(s * STRIPE + j * zr, zr)])
            return carry

        lax.fori_loop(0, STRIPE // zr, zs, 0)
        plsc.subcore_barrier()

        def body(g, carry):
            for t in range(3):
                j = g * 3 + t
                _gwait(j, t)
                _sstart(j, t)
                bn = (t + 2) % 3
                if t == 0:
                    @pl.when(g > 0)
                    def _():
                        _swait(g * 3 - 1, bn)

                    _gat(j + 2, bn)
                else:
                    @pl.when(g < ngrp - 1)
                    def _():
                        _swait(j - 1, bn)
                        _gat(j + 2, bn)
            return carry

        lax.fori_loop(0, ngrp, body, 0)
        _swait(CHUNKS - 3, 0)
        _swait(CHUNKS - 2, 1)
        _swait(CHUNKS - 1, 2)
        plsc.subcore_barrier()
        pltpu.sync_copy(acc.at[pl.ds(s * STRIPE, STRIPE)],
                        agg_out.at[CORES * p + c, pl.ds(s * STRIPE, STRIPE)])


def _make_agg(fh, passes):
    return pl.kernel(
        functools.partial(_agg_body, fh, passes),
        out_type=jax.ShapeDtypeStruct((CORES * passes, NP, fh), jnp.float32),
        mesh=_mesh(),
        scratch_types=[
            pltpu.VMEM((CHUNKS, CHUNK), jnp.int32),
            pltpu.VMEM((CHUNKS, CHUNK), jnp.int32),
            pltpu.VMEM((CHUNK, fh), jnp.float32),
            pltpu.VMEM((CHUNK, fh), jnp.float32),
            pltpu.VMEM((CHUNK, fh), jnp.float32),
            pltpu.VMEM((64, fh), jnp.float32),
            pltpu.VMEM_SHARED((NP, fh), jnp.float32),
            pltpu.SemaphoreType.DMA,
            pltpu.SemaphoreType.DMA,
            pltpu.SemaphoreType.DMA,
            pltpu.SemaphoreType.DMA,
            pltpu.SemaphoreType.DMA,
            pltpu.SemaphoreType.DMA,
        ],
        compiler_params=pltpu.CompilerParams(use_tc_tiling_on_sc=False),
    )


_DEG = _make_deg()
_AGGX = _make_agg(64, 1)
_AGG64 = _make_agg(64, 2)
_AGG32 = _make_agg(32, 1)


def _split_store(o_ref, y):
    ns = o_ref.shape[0]
    fh = y.shape[1] // ns
    for q in range(ns):
        o_ref[q] = y[:, q * fh:(q + 1) * fh]


def _merge(agg_ref, y_ref, dis):
    ns = agg_ref.shape[0]
    return jnp.concatenate(
        [(agg_ref[q] + y_ref[q]) * dis for q in range(ns)], axis=1)


def _xd_body(x_ref, d0_ref, d1_ref, o_ref):
    dis = lax.rsqrt(d0_ref[...] + d1_ref[...] + 1.0)
    _split_store(o_ref, x_ref[...] * dis)


def _mid1_body(aggx_ref, x_ref, d0_ref, d1_ref, b_ref, w1_ref, w2_ref, o_ref):
    dis = lax.rsqrt(d0_ref[...] + d1_ref[...] + 1.0)
    ns = aggx_ref.shape[0]
    fh = x_ref.shape[1] // ns
    u = jnp.concatenate([aggx_ref[q] for q in range(ns)], axis=1)
    u = u + x_ref[...] * dis
    h = jnp.dot(u, w1_ref[...], preferred_element_type=jnp.float32)
    h = jnp.maximum(h * dis + b_ref[...], 0.0)
    y = jnp.dot(h, w2_ref[...], preferred_element_type=jnp.float32) * dis
    _split_store(o_ref, y)


def _mid_body(agg_ref, y_ref, d0_ref, d1_ref, b_ref, w_ref, o_ref):
    dis = lax.rsqrt(d0_ref[...] + d1_ref[...] + 1.0)
    h = jnp.maximum(_merge(agg_ref, y_ref, dis) + b_ref[...], 0.0)
    y = jnp.dot(h, w_ref[...], preferred_element_type=jnp.float32) * dis
    _split_store(o_ref, y)


def _fin_body(agg_ref, y_ref, d0_ref, d1_ref, b_ref, o_ref):
    dis = lax.rsqrt(d0_ref[...] + d1_ref[...] + 1.0)
    z = _merge(agg_ref, y_ref, dis) + b_ref[...]
    z = z - jnp.max(z, axis=1, keepdims=True)
    e = jnp.exp(z)
    o_ref[...] = e / jnp.sum(e, axis=1, keepdims=True)


def _col_spec():
    return pl.BlockSpec((ROWS_BLK, 1), lambda i: (i, 0))


def _split_spec(ns, fh):
    return pl.BlockSpec((ns, ROWS_BLK, fh), lambda i: (0, i, 0))


def _xd(xp, d0, d1, osplit):
    fin = xp.shape[1]
    return pl.pallas_call(
        _xd_body,
        grid=(NBLK,),
        in_specs=[
            pl.BlockSpec((ROWS_BLK, fin), lambda i: (i, 0)),
            _col_spec(), _col_spec(),
        ],
        out_specs=_split_spec(osplit, fin // osplit),
        out_shape=jax.ShapeDtypeStruct((osplit, NP, fin // osplit), jnp.float32),
    )(xp, d0, d1)


def _mid1(aggx, xp, d0, d1, b, w1, w2, osplit):
    fin = xp.shape[1]
    fout = w2.shape[1]
    ins = aggx.shape[0]
    return pl.pallas_call(
        _mid1_body,
        grid=(NBLK,),
        in_specs=[
            _split_spec(ins, fin // ins),
            pl.BlockSpec((ROWS_BLK, fin), lambda i: (i, 0)),
            _col_spec(), _col_spec(),
            pl.BlockSpec((1, w1.shape[1]), lambda i: (0, 0)),
            pl.BlockSpec(w1.shape, lambda i: (0, 0)),
            pl.BlockSpec(w2.shape, lambda i: (0, 0)),
        ],
        out_specs=_split_spec(osplit, fout // osplit),
        out_shape=jax.ShapeDtypeStruct((osplit, NP, fout // osplit), jnp.float32),
    )(aggx, xp, d0, d1, b, w1, w2)


def _mid(agg, y, d0, d1, b, w, osplit):
    fin, fout = w.shape
    ins = agg.shape[0]
    return pl.pallas_call(
        _mid_body,
        grid=(NBLK,),
        in_specs=[
            _split_spec(ins, fin // ins),
            _split_spec(ins, fin // ins),
            _col_spec(), _col_spec(),
            pl.BlockSpec((1, fin), lambda i: (0, 0)),
            pl.BlockSpec((fin, fout), lambda i: (0, 0)),
        ],
        out_specs=_split_spec(osplit, fout // osplit),
        out_shape=jax.ShapeDtypeStruct((osplit, NP, fout // osplit), jnp.float32),
    )(agg, y, d0, d1, b, w)


def _fin(agg, y, d0, d1, b):
    fout = b.shape[1]
    ins = agg.shape[0]
    return pl.pallas_call(
        _fin_body,
        grid=(NBLK,),
        in_specs=[
            _split_spec(ins, fout // ins),
            _split_spec(ins, fout // ins),
            _col_spec(), _col_spec(),
            pl.BlockSpec((1, fout), lambda i: (0, 0)),
        ],
        out_specs=pl.BlockSpec((ROWS_BLK, fout), lambda i: (i, 0)),
        out_shape=jax.ShapeDtypeStruct((NP, fout), jnp.float32),
    )(agg, y, d0, d1, b)


def kernel(x, edge_index, W1, b1, W2, b2, W3, b3):
    src = edge_index[0].astype(jnp.int32)
    dst = edge_index[1].astype(jnp.int32)
    npad = EPAD - E
    fill = jnp.arange(npad, dtype=jnp.int32) % 8
    src_t = jnp.concatenate([src, fill]).reshape(TILES, CHUNKS, CHUNK)
    dst_t = jnp.concatenate([dst, N + fill]).reshape(TILES, CHUNKS, CHUNK)
    srcoff = jnp.stack([src_t, src_t + NP])
    dfill = jnp.arange(DPAD - E, dtype=jnp.int32) % 8
    dst_d = jnp.concatenate([dst, N + dfill]).reshape(CORES, TILES, DCHUNKS, CHUNK)

    xp = jnp.pad(x, ((0, NP - N), (0, 0)))

    deg = _DEG(dst_d)
    d0 = deg[0].reshape(NP, 1)
    d1 = deg[1].reshape(NP, 1)

    xd = _xd(xp, d0, d1, 2)
    aggx = _AGGX(xd.reshape(2 * NP, 64), srcoff, dst_t)
    y2 = _mid1(aggx, xp, d0, d1, b1.reshape(1, -1), W1, W2, 4)
    agg2 = _AGG64(y2.reshape(4 * NP, 64), srcoff, dst_t)
    y3 = _mid(agg2, y2, d0, d1, b2.reshape(1, -1), W3, 2)
    agg3 = _AGG32(y3.reshape(2 * NP, 32), srcoff, dst_t)
    out = _fin(agg3, y3, d0, d1, b3.reshape(1, -1))
    return out[:N]


# speedup vs baseline: 1.4569x; 1.0156x over previous
import functools

import jax
import jax.numpy as jnp
from jax import lax
from jax.experimental import pallas as pl
from jax.experimental.pallas import tpu as pltpu
from jax.experimental.pallas import tpu_sc as plsc

N = 10000
NP = 10240
E = 320000
TILES = 16
CORES = 2
CHUNK = 128
CHUNKS = 159
EPAD = TILES * CHUNKS * CHUNK
DCHUNKS = 80
DPAD = CORES * TILES * DCHUNKS * CHUNK
STRIPE = NP // TILES
ROWS_BLK = 512
NBLK = NP // ROWS_BLK


def _mesh():
    return plsc.VectorSubcoreMesh(core_axis_name="c", subcore_axis_name="s")


def _deg_body(dst_hbm, deg_out, idx_d, ones_v, zbuf, acc):
    c = lax.axis_index("c")
    s = lax.axis_index("s")

    def zb(i, carry):
        zbuf[pl.ds(i * 16, 16)] = jnp.zeros((16,), jnp.float32)
        return carry

    lax.fori_loop(0, STRIPE // 16, zb, 0)
    for i in range(CHUNK // 16):
        ones_v[pl.ds(i * 16, 16)] = jnp.ones((16,), jnp.float32)
    pltpu.sync_copy(zbuf, acc.at[pl.ds(s * STRIPE, STRIPE)])
    pltpu.sync_copy(dst_hbm.at[s], idx_d)
    plsc.subcore_barrier()

    def body(j, carry):
        pltpu.sync_copy(ones_v, acc.at[idx_d.at[j]], add=True)
        return carry

    lax.fori_loop(c * (CHUNKS // 2 + 1), (CHUNKS // 2 + 1) + c * (CHUNKS // 2),
                  body, 0)
    plsc.subcore_barrier()
    pltpu.sync_copy(acc.at[pl.ds(s * STRIPE, STRIPE)],
                    deg_out.at[c, pl.ds(s * STRIPE, STRIPE)])


def _make_deg():
    return pl.kernel(
        _deg_body,
        out_type=jax.ShapeDtypeStruct((CORES, NP), jnp.float32),
        mesh=_mesh(),
        scratch_types=[
            pltpu.VMEM((CHUNKS, CHUNK), jnp.int32),
            pltpu.VMEM((CHUNK,), jnp.float32),
            pltpu.VMEM((STRIPE,), jnp.float32),
            pltpu.VMEM_SHARED((NP,), jnp.float32),
        ],
        compiler_params=pltpu.CompilerParams(use_tc_tiling_on_sc=False),
    )


def _agg_body(fh, passes, y_hbm, srcoff_hbm, dst_hbm, agg_out,
              idx_s, idx_d, r0, r1, r2, zbuf, acc,
              g0, g1, g2, s0, s1, s2):
    rows = [r0, r1, r2]
    gsem = [g0, g1, g2]
    ssem = [s0, s1, s2]
    c = lax.axis_index("c")
    s = lax.axis_index("s")
    zr = zbuf.shape[0]
    ngrp = CHUNKS // 3

    def zb(i, carry):
        for k in range(fh // 16):
            zbuf[i, pl.ds(k * 16, 16)] = jnp.zeros((16,), jnp.float32)
        return carry

    lax.fori_loop(0, zr, zb, 0)
    pltpu.sync_copy(srcoff_hbm.at[c, s], idx_s)
    pltpu.sync_copy(dst_hbm.at[s], idx_d)

    def _gat(j, b):
        pltpu.async_copy(y_hbm.at[idx_s.at[j]], rows[b], gsem[b])

    def _gwait(j, b):
        pltpu.make_async_copy(y_hbm.at[idx_s.at[j]], rows[b], gsem[b]).wait()

    def _sstart(j, b):
        pltpu.async_copy(rows[b], acc.at[idx_d.at[j]], ssem[b], add=True)

    def _swait(j, b):
        pltpu.make_async_copy(rows[b], acc.at[idx_d.at[j]], ssem[b]).wait()

    for p in range(passes):
        if p > 0:
            def shift(i, carry):
                for k in range(CHUNK // 16):
                    idx_s[i, pl.ds(k * 16, 16)] = (
                        idx_s[i, pl.ds(k * 16, 16)] + CORES * NP)
                return carry

            lax.fori_loop(0, CHUNKS, shift, 0)

        _gat(0, 0)
        _gat(1, 1)

        def zs(j, carry):
            pltpu.sync_copy(zbuf, acc.at[pl.ds(s * STRIPE + j * zr, zr)])
            return carry

        lax.fori_loop(0, STRIPE // zr, zs, 0)
        plsc.subcore_barrier()

        def body(g, carry):
            for t in range(3):
                j = g * 3 + t
                _gwait(j, t)
                _sstart(j, t)
                bn = (t + 2) % 3
                if t == 0:
                    @pl.when(g > 0)
                    def _():
                        _swait(g * 3 - 1, bn)

                    _gat(j + 2, bn)
                else:
                    @pl.when(g < ngrp - 1)
                    def _():
                        _swait(j - 1, bn)
                        _gat(j + 2, bn)
            return carry

        lax.fori_loop(0, ngrp, body, 0)
        _swait(CHUNKS - 3, 0)
        _swait(CHUNKS - 2, 1)
        _swait(CHUNKS - 1, 2)
        plsc.subcore_barrier()
        pltpu.sync_copy(acc.at[pl.ds(s * STRIPE, STRIPE)],
                        agg_out.at[CORES * p + c, pl.ds(s * STRIPE, STRIPE)])


def _make_agg(fh, passes):
    return pl.kernel(
        functools.partial(_agg_body, fh, passes),
        out_type=jax.ShapeDtypeStruct((CORES * passes, NP, fh), jnp.float32),
        mesh=_mesh(),
        scratch_types=[
            pltpu.VMEM((CHUNKS, CHUNK), jnp.int32),
            pltpu.VMEM((CHUNKS, CHUNK), jnp.int32),
            pltpu.VMEM((CHUNK, fh), jnp.float32),
            pltpu.VMEM((CHUNK, fh), jnp.float32),
            pltpu.VMEM((CHUNK, fh), jnp.float32),
            pltpu.VMEM((64, fh), jnp.float32),
            pltpu.VMEM_SHARED((NP, fh), jnp.float32),
            pltpu.SemaphoreType.DMA,
            pltpu.SemaphoreType.DMA,
            pltpu.SemaphoreType.DMA,
            pltpu.SemaphoreType.DMA,
            pltpu.SemaphoreType.DMA,
            pltpu.SemaphoreType.DMA,
        ],
        compiler_params=pltpu.CompilerParams(use_tc_tiling_on_sc=False),
    )


_DEG = _make_deg()
_AGGX = _make_agg(64, 1)
_AGG64 = _make_agg(64, 2)
_AGG32 = _make_agg(32, 1)


def _split_store(o_ref, y):
    ns = o_ref.shape[0]
    fh = y.shape[1] // ns
    for q in range(ns):
        o_ref[q] = y[:, q * fh:(q + 1) * fh]


def _merge(agg_ref, y_ref, dis):
    ns = agg_ref.shape[0]
    return jnp.concatenate(
        [(agg_ref[q] + y_ref[q]) * dis for q in range(ns)], axis=1)


def _xd_body(x_ref, d0_ref, d1_ref, o_ref):
    dis = lax.rsqrt(d0_ref[...] + d1_ref[...] + 1.0)
    _split_store(o_ref, x_ref[...] * dis)


def _mid1_body(aggx_ref, x_ref, d0_ref, d1_ref, b_ref, w1_ref, w2_ref, o_ref):
    dis = lax.rsqrt(d0_ref[...] + d1_ref[...] + 1.0)
    ns = aggx_ref.shape[0]
    fh = x_ref.shape[1] // ns
    u = jnp.concatenate([aggx_ref[q] for q in range(ns)], axis=1)
    u = u + x_ref[...] * dis
    h = jnp.dot(u, w1_ref[...], preferred_element_type=jnp.float32)
    h = jnp.maximum(h * dis + b_ref[...], 0.0)
    y = jnp.dot(h, w2_ref[...], preferred_element_type=jnp.float32) * dis
    _split_store(o_ref, y)


def _mid_body(agg_ref, y_ref, d0_ref, d1_ref, b_ref, w_ref, o_ref):
    dis = lax.rsqrt(d0_ref[...] + d1_ref[...] + 1.0)
    h = jnp.maximum(_merge(agg_ref, y_ref, dis) + b_ref[...], 0.0)
    y = jnp.dot(h, w_ref[...], preferred_element_type=jnp.float32) * dis
    _split_store(o_ref, y)


def _fin_body(agg_ref, y_ref, d0_ref, d1_ref, b_ref, o_ref):
    dis = lax.rsqrt(d0_ref[...] + d1_ref[...] + 1.0)
    z = _merge(agg_ref, y_ref, dis) + b_ref[...]
    z = z - jnp.max(z, axis=1, keepdims=True)
    e = jnp.exp(z)
    o_ref[...] = e / jnp.sum(e, axis=1, keepdims=True)


def _col_spec():
    return pl.BlockSpec((ROWS_BLK, 1), lambda i: (i, 0))


def _split_spec(ns, fh):
    return pl.BlockSpec((ns, ROWS_BLK, fh), lambda i: (0, i, 0))


def _xd(xp, d0, d1, osplit):
    fin = xp.shape[1]
    return pl.pallas_call(
        _xd_body,
        grid=(NBLK,),
        in_specs=[
            pl.BlockSpec((ROWS_BLK, fin), lambda i: (i, 0)),
            _col_spec(), _col_spec(),
        ],
        out_specs=_split_spec(osplit, fin // osplit),
        out_shape=jax.ShapeDtypeStruct((osplit, NP, fin // osplit), jnp.float32),
    )(xp, d0, d1)


def _mid1(aggx, xp, d0, d1, b, w1, w2, osplit):
    fin = xp.shape[1]
    fout = w2.shape[1]
    ins = aggx.shape[0]
    return pl.pallas_call(
        _mid1_body,
        grid=(NBLK,),
        in_specs=[
            _split_spec(ins, fin // ins),
            pl.BlockSpec((ROWS_BLK, fin), lambda i: (i, 0)),
            _col_spec(), _col_spec(),
            pl.BlockSpec((1, w1.shape[1]), lambda i: (0, 0)),
            pl.BlockSpec(w1.shape, lambda i: (0, 0)),
            pl.BlockSpec(w2.shape, lambda i: (0, 0)),
        ],
        out_specs=_split_spec(osplit, fout // osplit),
        out_shape=jax.ShapeDtypeStruct((osplit, NP, fout // osplit), jnp.float32),
    )(aggx, xp, d0, d1, b, w1, w2)


def _mid(agg, y, d0, d1, b, w, osplit):
    fin, fout = w.shape
    ins = agg.shape[0]
    return pl.pallas_call(
        _mid_body,
        grid=(NBLK,),
        in_specs=[
            _split_spec(ins, fin // ins),
            _split_spec(ins, fin // ins),
            _col_spec(), _col_spec(),
            pl.BlockSpec((1, fin), lambda i: (0, 0)),
            pl.BlockSpec((fin, fout), lambda i: (0, 0)),
        ],
        out_specs=_split_spec(osplit, fout // osplit),
        out_shape=jax.ShapeDtypeStruct((osplit, NP, fout // osplit), jnp.float32),
    )(agg, y, d0, d1, b, w)


def _fin(agg, y, d0, d1, b):
    fout = b.shape[1]
    ins = agg.shape[0]
    return pl.pallas_call(
        _fin_body,
        grid=(NBLK,),
        in_specs=[
            _split_spec(ins, fout // ins),
            _split_spec(ins, fout // ins),
            _col_spec(), _col_spec(),
            pl.BlockSpec((1, fout), lambda i: (0, 0)),
        ],
        out_specs=pl.BlockSpec((ROWS_BLK, fout), lambda i: (i, 0)),
        out_shape=jax.ShapeDtypeStruct((NP, fout), jnp.float32),
    )(agg, y, d0, d1, b)


def kernel(x, edge_index, W1, b1, W2, b2, W3, b3):
    src = edge_index[0].astype(jnp.int32)
    dst = edge_index[1].astype(jnp.int32)
    npad = EPAD - E
    fill = jnp.arange(npad, dtype=jnp.int32) % 8
    src_t = jnp.concatenate([src, fill]).reshape(TILES, CHUNKS, CHUNK)
    dst_t = jnp.concatenate([dst, N + fill]).reshape(TILES, CHUNKS, CHUNK)
    srcoff = jnp.stack([src_t, src_t + NP])

    xp = jnp.pad(x, ((0, NP - N), (0, 0)))

    deg = _DEG(dst_t)
    d0 = deg[0].reshape(NP, 1)
    d1 = deg[1].reshape(NP, 1)

    xd = _xd(xp, d0, d1, 2)
    aggx = _AGGX(xd.reshape(2 * NP, 64), srcoff, dst_t)
    y2 = _mid1(aggx, xp, d0, d1, b1.reshape(1, -1), W1, W2, 4)
    agg2 = _AGG64(y2.reshape(4 * NP, 64), srcoff, dst_t)
    y3 = _mid(agg2, y2, d0, d1, b2.reshape(1, -1), W3, 2)
    agg3 = _AGG32(y3.reshape(2 * NP, 32), srcoff, dst_t)
    out = _fin(agg3, y3, d0, d1, b3.reshape(1, -1))
    return out[:N]


# speedup vs baseline: 1.4574x; 1.0003x over previous
import functools

import jax
import jax.numpy as jnp
from jax import lax
from jax.experimental import pallas as pl
from jax.experimental.pallas import tpu as pltpu
from jax.experimental.pallas import tpu_sc as plsc

N = 10000
NP = 10240
E = 320000
TILES = 16
CORES = 2
CHUNK = 128
CHUNKS = 159
EPAD = TILES * CHUNKS * CHUNK
STRIPE = NP // TILES
ROWS_BLK = 512
NBLK = NP // ROWS_BLK


def _mesh():
    return plsc.VectorSubcoreMesh(core_axis_name="c", subcore_axis_name="s")


def _deg_body(dst_hbm, deg_out, idx_d, ones_v, zbuf, acc):
    c = lax.axis_index("c")
    s = lax.axis_index("s")

    def zb(i, carry):
        zbuf[pl.ds(i * 16, 16)] = jnp.zeros((16,), jnp.float32)
        return carry

    lax.fori_loop(0, STRIPE // 16, zb, 0)
    for i in range(CHUNK // 16):
        ones_v[pl.ds(i * 16, 16)] = jnp.ones((16,), jnp.float32)
    pltpu.sync_copy(zbuf, acc.at[pl.ds(s * STRIPE, STRIPE)])
    pltpu.sync_copy(dst_hbm.at[s], idx_d)
    plsc.subcore_barrier()

    def body(j, carry):
        pltpu.sync_copy(ones_v, acc.at[idx_d.at[j]], add=True)
        return carry

    lax.fori_loop(c * (CHUNKS // 2 + 1), (CHUNKS // 2 + 1) + c * (CHUNKS // 2),
                  body, 0)
    plsc.subcore_barrier()
    pltpu.sync_copy(acc.at[pl.ds(s * STRIPE, STRIPE)],
                    deg_out.at[c, pl.ds(s * STRIPE, STRIPE)])


def _make_deg():
    return pl.kernel(
        _deg_body,
        out_type=jax.ShapeDtypeStruct((CORES, NP), jnp.float32),
        mesh=_mesh(),
        scratch_types=[
            pltpu.VMEM((CHUNKS, CHUNK), jnp.int32),
            pltpu.VMEM((CHUNK,), jnp.float32),
            pltpu.VMEM((STRIPE,), jnp.float32),
            pltpu.VMEM_SHARED((NP,), jnp.float32),
        ],
        compiler_params=pltpu.CompilerParams(use_tc_tiling_on_sc=False),
    )


def _agg_body(fh, passes, y_hbm, srcoff_hbm, dst_hbm, agg_out,
              idx_s, idx_d, r0, r1, r2, zbuf, acc,
              g0, g1, g2, s0, s1, s2):
    rows = [r0, r1, r2]
    gsem = [g0, g1, g2]
    ssem = [s0, s1, s2]
    c = lax.axis_index("c")
    s = lax.axis_index("s")
    zr = zbuf.shape[0]
    ngrp = CHUNKS // 3

    def zb(i, carry):
        for k in range(fh // 16):
            zbuf[i, pl.ds(k * 16, 16)] = jnp.zeros((16,), jnp.float32)
        return carry

    lax.fori_loop(0, zr, zb, 0)
    pltpu.sync_copy(srcoff_hbm.at[c, s], idx_s)
    pltpu.sync_copy(dst_hbm.at[s], idx_d)

    def _gat(j, b):
        pltpu.async_copy(y_hbm.at[idx_s.at[j]], rows[b], gsem[b])

    def _gwait(j, b):
        pltpu.make_async_copy(y_hbm.at[idx_s.at[j]], rows[b], gsem[b]).wait()

    def _sstart(j, b):
        pltpu.async_copy(rows[b], acc.at[idx_d.at[j]], ssem[b], add=True)

    def _swait(j, b):
        pltpu.make_async_copy(rows[b], acc.at[idx_d.at[j]], ssem[b]).wait()

    for p in range(passes):
        if p > 0:
            def shift(i, carry):
                for k in range(CHUNK // 16):
                    idx_s[i, pl.ds(k * 16, 16)] = (
                        idx_s[i, pl.ds(k * 16, 16)] + CORES * NP)
                return carry

            lax.fori_loop(0, CHUNKS, shift, 0)

        _gat(0, 0)
        _gat(1, 1)

        def zs(j, carry):
            pltpu.sync_copy(zbuf, acc.at[pl.ds(s * STRIPE + j * zr, zr)])
            return carry

        lax.fori_loop(0, STRIPE // zr, zs, 0)
        plsc.subcore_barrier()

        def body(g, carry):
            for t in range(3):
                j = g * 3 + t
                _gwait(j, t)
                _sstart(j, t)
                bn = (t + 2) % 3
                if t == 0:
                    @pl.when(g > 0)
                    def _():
                        _swait(g * 3 - 1, bn)

                    _gat(j + 2, bn)
                else:
                    @pl.when(g < ngrp - 1)
                    def _():
                        _swait(j - 1, bn)
                        _gat(j + 2, bn)
            return carry

        lax.fori_loop(0, ngrp, body, 0)
        _swait(CHUNKS - 3, 0)
        _swait(CHUNKS - 2, 1)
        _swait(CHUNKS - 1, 2)
        plsc.subcore_barrier()
        pltpu.sync_copy(acc.at[pl.ds(s * STRIPE, STRIPE)],
                        agg_out.at[CORES * p + c, pl.ds(s * STRIPE, STRIPE)])


def _make_agg(fh, passes):
    return pl.kernel(
        functools.partial(_agg_body, fh, passes),
        out_type=jax.ShapeDtypeStruct((CORES * passes, NP, fh), jnp.float32),
        mesh=_mesh(),
        scratch_types=[
            pltpu.VMEM((CHUNKS, CHUNK), jnp.int32),
            pltpu.VMEM((CHUNKS, CHUNK), jnp.int32),
            pltpu.VMEM((CHUNK, fh), jnp.float32),
            pltpu.VMEM((CHUNK, fh), jnp.float32),
            pltpu.VMEM((CHUNK, fh), jnp.float32),
            pltpu.VMEM((64, fh), jnp.float32),
            pltpu.VMEM_SHARED((NP, fh), jnp.float32),
            pltpu.SemaphoreType.DMA,
            pltpu.SemaphoreType.DMA,
            pltpu.SemaphoreType.DMA,
            pltpu.SemaphoreType.DMA,
            pltpu.SemaphoreType.DMA,
            pltpu.SemaphoreType.DMA,
        ],
        compiler_params=pltpu.CompilerParams(use_tc_tiling_on_sc=False),
    )


_DEG = _make_deg()
_AGGX = _make_agg(64, 1)
_AGG64 = _make_agg(64, 2)
_AGG32 = _make_agg(32, 1)


def _split_store(o_ref, y):
    ns = o_ref.shape[0]
    fh = y.shape[1] // ns
    for q in range(ns):
        o_ref[q] = y[:, q * fh:(q + 1) * fh]


def _merge(agg_ref, y_ref, dis):
    ns = agg_ref.shape[0]
    return jnp.concatenate(
        [(agg_ref[q] + y_ref[q]) * dis for q in range(ns)], axis=1)


def _xd_body(x_ref, d0_ref, d1_ref, o_ref):
    dis = lax.rsqrt(d0_ref[...] + d1_ref[...] + 1.0)
    _split_store(o_ref, x_ref[...] * dis)


def _mid1_body(aggx_ref, x_ref, d0_ref, d1_ref, b_ref, w1_ref, w2_ref, o_ref):
    dis = lax.rsqrt(d0_ref[...] + d1_ref[...] + 1.0)
    ns = aggx_ref.shape[0]
    u = jnp.concatenate([aggx_ref[q] for q in range(ns)], axis=1)
    u = u + x_ref[...] * dis
    h = jnp.dot(u, w1_ref[...], preferred_element_type=jnp.float32)
    h = jnp.maximum(h * dis + b_ref[...], 0.0)
    y = jnp.dot(h, w2_ref[...], preferred_element_type=jnp.float32) * dis
    _split_store(o_ref, y)


def _mid_body(agg_ref, y_ref, d0_ref, d1_ref, b_ref, w_ref, o_ref):
    dis = lax.rsqrt(d0_ref[...] + d1_ref[...] + 1.0)
    h = jnp.maximum(_merge(agg_ref, y_ref, dis) + b_ref[...], 0.0)
    y = jnp.dot(h, w_ref[...], preferred_element_type=jnp.float32) * dis
    _split_store(o_ref, y)


def _fin_body(agg_ref, y_ref, d0_ref, d1_ref, b_ref, o_ref):
    dis = lax.rsqrt(d0_ref[...] + d1_ref[...] + 1.0)
    z = _merge(agg_ref, y_ref, dis) + b_ref[...]
    z = z - jnp.max(z, axis=1, keepdims=True)
    e = jnp.exp(z)
    o_ref[...] = e / jnp.sum(e, axis=1, keepdims=True)


def _col_spec():
    return pl.BlockSpec((ROWS_BLK, 1), lambda i: (i, 0))


def _split_spec(ns, fh):
    return pl.BlockSpec((ns, ROWS_BLK, fh), lambda i: (0, i, 0))


def _xd(xp, d0, d1, osplit):
    fin = xp.shape[1]
    return pl.pallas_call(
        _xd_body,
        grid=(NBLK,),
        in_specs=[
            pl.BlockSpec((ROWS_BLK, fin), lambda i: (i, 0)),
            _col_spec(), _col_spec(),
        ],
        out_specs=_split_spec(osplit, fin // osplit),
        out_shape=jax.ShapeDtypeStruct((osplit, NP, fin // osplit), jnp.float32),
    )(xp, d0, d1)


def _mid1(aggx, xp, d0, d1, b, w1, w2, osplit):
    fin = xp.shape[1]
    fout = w2.shape[1]
    ins = aggx.shape[0]
    return pl.pallas_call(
        _mid1_body,
        grid=(NBLK,),
        in_specs=[
            _split_spec(ins, fin // ins),
            pl.BlockSpec((ROWS_BLK, fin), lambda i: (i, 0)),
            _col_spec(), _col_spec(),
            pl.BlockSpec((1, w1.shape[1]), lambda i: (0, 0)),
            pl.BlockSpec(w1.shape, lambda i: (0, 0)),
            pl.BlockSpec(w2.shape, lambda i: (0, 0)),
        ],
        out_specs=_split_spec(osplit, fout // osplit),
        out_shape=jax.ShapeDtypeStruct((osplit, NP, fout // osplit), jnp.float32),
    )(aggx, xp, d0, d1, b, w1, w2)


def _mid(agg, y, d0, d1, b, w, osplit):
    fin, fout = w.shape
    ins = agg.shape[0]
    return pl.pallas_call(
        _mid_body,
        grid=(NBLK,),
        in_specs=[
            _split_spec(ins, fin // ins),
            _split_spec(ins, fin // ins),
            _col_spec(), _col_spec(),
            pl.BlockSpec((1, fin), lambda i: (0, 0)),
            pl.BlockSpec((fin, fout), lambda i: (0, 0)),
        ],
        out_specs=_split_spec(osplit, fout // osplit),
        out_shape=jax.ShapeDtypeStruct((osplit, NP, fout // osplit), jnp.float32),
    )(agg, y, d0, d1, b, w)


def _fin(agg, y, d0, d1, b):
    fout = b.shape[1]
    ins = agg.shape[0]
    return pl.pallas_call(
        _fin_body,
        grid=(NBLK,),
        in_specs=[
            _split_spec(ins, fout // ins),
            _split_spec(ins, fout // ins),
            _col_spec(), _col_spec(),
            pl.BlockSpec((1, fout), lambda i: (0, 0)),
        ],
        out_specs=pl.BlockSpec((ROWS_BLK, fout), lambda i: (i, 0)),
        out_shape=jax.ShapeDtypeStruct((NP, fout), jnp.float32),
    )(agg, y, d0, d1, b)


def kernel(x, edge_index, W1, b1, W2, b2, W3, b3):
    src = edge_index[0].astype(jnp.int32)
    dst = edge_index[1].astype(jnp.int32)
    npad = EPAD - E
    fill = jnp.arange(npad, dtype=jnp.int32) % 8
    src_t = jnp.concatenate([src, fill]).reshape(TILES, CHUNKS, CHUNK)
    dst_t = jnp.concatenate([dst, N + fill]).reshape(TILES, CHUNKS, CHUNK)
    srcoff = jnp.stack([src_t, src_t + NP])

    xp = jnp.pad(x, ((0, NP - N), (0, 0)))

    deg = _DEG(dst_t)
    d0 = deg[0].reshape(NP, 1)
    d1 = deg[1].reshape(NP, 1)

    xd = _xd(xp, d0, d1, 2)
    aggx = _AGGX(xd.reshape(2 * NP, 64), srcoff, dst_t)
    y2 = _mid1(aggx, xp, d0, d1, b1.reshape(1, -1), W1, W2, 4)
    agg2 = _AGG64(y2.reshape(4 * NP, 64), srcoff, dst_t)
    y3 = _mid(agg2, y2, d0, d1, b2.reshape(1, -1), W3, 2)
    agg3 = _AGG32(y3.reshape(2 * NP, 32), srcoff, dst_t)
    out = _fin(agg3, y3, d0, d1, b3.reshape(1, -1))
    return out[:N]


# speedup vs baseline: 1.4579x; 1.0004x over previous
import functools

import jax
import jax.numpy as jnp
from jax import lax
from jax.experimental import pallas as pl
from jax.experimental.pallas import tpu as pltpu
from jax.experimental.pallas import tpu_sc as plsc

N = 10000
NP = 10240
E = 320000
TILES = 16
CORES = 2
CHUNK = 128
CHUNKS = 159
EPAD = TILES * CHUNKS * CHUNK
STRIPE = NP // TILES
ROWS_BLK = 512
NBLK = NP // ROWS_BLK


def _mesh():
    return plsc.VectorSubcoreMesh(core_axis_name="c", subcore_axis_name="s",
                                  num_cores=CORES, num_subcores=TILES)


def _deg_body(dst_hbm, deg_out, idx_d, ones_v, zbuf, acc):
    c = lax.axis_index("c")
    s = lax.axis_index("s")

    def zb(i, carry):
        zbuf[pl.ds(i * 16, 16)] = jnp.zeros((16,), jnp.float32)
        return carry

    lax.fori_loop(0, STRIPE // 16, zb, 0)
    for i in range(CHUNK // 16):
        ones_v[pl.ds(i * 16, 16)] = jnp.ones((16,), jnp.float32)
    pltpu.sync_copy(zbuf, acc.at[pl.ds(s * STRIPE, STRIPE)])
    pltpu.sync_copy(dst_hbm.at[s], idx_d)
    plsc.subcore_barrier()

    def body(j, carry):
        pltpu.sync_copy(ones_v, acc.at[idx_d.at[j]], add=True)
        return carry

    lax.fori_loop(c * (CHUNKS // 2 + 1), (CHUNKS // 2 + 1) + c * (CHUNKS // 2),
                  body, 0)
    plsc.subcore_barrier()
    pltpu.sync_copy(acc.at[pl.ds(s * STRIPE, STRIPE)],
                    deg_out.at[c, pl.ds(s * STRIPE, STRIPE)])


def _make_deg():
    return pl.kernel(
        _deg_body,
        out_type=jax.ShapeDtypeStruct((CORES, NP), jnp.float32),
        mesh=_mesh(),
        scratch_types=[
            pltpu.VMEM((CHUNKS, CHUNK), jnp.int32),
            pltpu.VMEM((CHUNK,), jnp.float32),
            pltpu.VMEM((STRIPE,), jnp.float32),
            pltpu.VMEM_SHARED((NP,), jnp.float32),
        ],
        compiler_params=pltpu.CompilerParams(use_tc_tiling_on_sc=False),
    )


def _agg_body(fh, passes, y_hbm, srcoff_hbm, dst_hbm, agg_out,
              idx_s, idx_d, r0, r1, r2, zbuf, acc,
              g0, g1, g2, s0, s1, s2):
    rows = [r0, r1, r2]
    gsem = [g0, g1, g2]
    ssem = [s0, s1, s2]
    c = lax.axis_index("c")
    s = lax.axis_index("s")
    zr = zbuf.shape[0]
    ngrp = CHUNKS // 3

    def zb(i, carry):
        for k in range(fh // 16):
            zbuf[i, pl.ds(k * 16, 16)] = jnp.zeros((16,), jnp.float32)
        return carry

    lax.fori_loop(0, zr, zb, 0)
    pltpu.sync_copy(srcoff_hbm.at[c, s], idx_s)
    pltpu.sync_copy(dst_hbm.at[s], idx_d)

    def _gat(j, b):
        pltpu.async_copy(y_hbm.at[idx_s.at[j]], rows[b], gsem[b])

    def _gwait(j, b):
        pltpu.make_async_copy(y_hbm.at[idx_s.at[j]], rows[b], gsem[b]).wait()

    def _sstart(j, b):
        pltpu.async_copy(rows[b], acc.at[idx_d.at[j]], ssem[b], add=True)

    def _swait(j, b):
        pltpu.make_async_copy(rows[b], acc.at[idx_d.at[j]], ssem[b]).wait()

    for p in range(passes):
        if p > 0:
            def shift(i, carry):
                for k in range(CHUNK // 16):
                    idx_s[i, pl.ds(k * 16, 16)] = (
                        idx_s[i, pl.ds(k * 16, 16)] + CORES * NP)
                return carry

            lax.fori_loop(0, CHUNKS, shift, 0)

        _gat(0, 0)
        _gat(1, 1)

        def zs(j, carry):
            pltpu.sync_copy(zbuf, acc.at[pl.ds(s * STRIPE + j * zr, zr)])
            return carry

        lax.fori_loop(0, STRIPE // zr, zs, 0)
        plsc.subcore_barrier()

        def body(g, carry):
            for t in range(3):
                j = g * 3 + t
                _gwait(j, t)
                _sstart(j, t)
                bn = (t + 2) % 3
                if t == 0:
                    @pl.when(g > 0)
                    def _():
                        _swait(g * 3 - 1, bn)

                    _gat(j + 2, bn)
                else:
                    @pl.when(g < ngrp - 1)
                    def _():
                        _swait(j - 1, bn)
                        _gat(j + 2, bn)
            return carry

        lax.fori_loop(0, ngrp, body, 0)
        _swait(CHUNKS - 3, 0)
        _swait(CHUNKS - 2, 1)
        _swait(CHUNKS - 1, 2)
        plsc.subcore_barrier()
        pltpu.sync_copy(acc.at[pl.ds(s * STRIPE, STRIPE)],
                        agg_out.at[CORES * p + c, pl.ds(s * STRIPE, STRIPE)])


def _make_agg(fh, passes):
    return pl.kernel(
        functools.partial(_agg_body, fh, passes),
        out_type=jax.ShapeDtypeStruct((CORES * passes, NP, fh), jnp.float32),
        mesh=_mesh(),
        scratch_types=[
            pltpu.VMEM((CHUNKS, CHUNK), jnp.int32),
            pltpu.VMEM((CHUNKS, CHUNK), jnp.int32),
            pltpu.VMEM((CHUNK, fh), jnp.float32),
            pltpu.VMEM((CHUNK, fh), jnp.float32),
            pltpu.VMEM((CHUNK, fh), jnp.float32),
            pltpu.VMEM((64, fh), jnp.float32),
            pltpu.VMEM_SHARED((NP, fh), jnp.float32),
            pltpu.SemaphoreType.DMA,
            pltpu.SemaphoreType.DMA,
            pltpu.SemaphoreType.DMA,
            pltpu.SemaphoreType.DMA,
            pltpu.SemaphoreType.DMA,
            pltpu.SemaphoreType.DMA,
        ],
        compiler_params=pltpu.CompilerParams(use_tc_tiling_on_sc=False),
    )


_DEG = _make_deg()
_AGGX = _make_agg(64, 1)
_AGG64 = _make_agg(64, 2)
_AGG32 = _make_agg(32, 1)


def _split_store(o_ref, y):
    ns = o_ref.shape[0]
    fh = y.shape[1] // ns
    for q in range(ns):
        o_ref[q] = y[:, q * fh:(q + 1) * fh]


def _merge(agg_ref, y_ref, dis):
    ns = agg_ref.shape[0]
    return jnp.concatenate(
        [(agg_ref[q] + y_ref[q]) * dis for q in range(ns)], axis=1)


def _xd_body(x_ref, d0_ref, d1_ref, o_ref):
    dis = lax.rsqrt(d0_ref[...] + d1_ref[...] + 1.0)
    _split_store(o_ref, x_ref[...] * dis)


def _mid1_body(aggx_ref, x_ref, d0_ref, d1_ref, b_ref, w1_ref, w2_ref, o_ref):
    dis = lax.rsqrt(d0_ref[...] + d1_ref[...] + 1.0)
    ns = aggx_ref.shape[0]
    u = jnp.concatenate([aggx_ref[q] for q in range(ns)], axis=1)
    u = u + x_ref[...] * dis
    h = jnp.dot(u, w1_ref[...], preferred_element_type=jnp.float32)
    h = jnp.maximum(h * dis + b_ref[...], 0.0)
    y = jnp.dot(h, w2_ref[...], preferred_element_type=jnp.float32) * dis
    _split_store(o_ref, y)


def _mid_body(agg_ref, y_ref, d0_ref, d1_ref, b_ref, w_ref, o_ref):
    dis = lax.rsqrt(d0_ref[...] + d1_ref[...] + 1.0)
    h = jnp.maximum(_merge(agg_ref, y_ref, dis) + b_ref[...], 0.0)
    y = jnp.dot(h, w_ref[...], preferred_element_type=jnp.float32) * dis
    _split_store(o_ref, y)


def _fin_body(agg_ref, y_ref, d0_ref, d1_ref, b_ref, o_ref):
    dis = lax.rsqrt(d0_ref[...] + d1_ref[...] + 1.0)
    z = _merge(agg_ref, y_ref, dis) + b_ref[...]
    z = z - jnp.max(z, axis=1, keepdims=True)
    e = jnp.exp(z)
    o_ref[...] = e / jnp.sum(e, axis=1, keepdims=True)


def _col_spec():
    return pl.BlockSpec((ROWS_BLK, 1), lambda i: (i, 0))


def _split_spec(ns, fh):
    return pl.BlockSpec((ns, ROWS_BLK, fh), lambda i: (0, i, 0))


def _xd(xp, d0, d1, osplit):
    fin = xp.shape[1]
    return pl.pallas_call(
        _xd_body,
        grid=(NBLK,),
        in_specs=[
            pl.BlockSpec((ROWS_BLK, fin), lambda i: (i, 0)),
            _col_spec(), _col_spec(),
        ],
        out_specs=_split_spec(osplit, fin // osplit),
        out_shape=jax.ShapeDtypeStruct((osplit, NP, fin // osplit), jnp.float32),
    )(xp, d0, d1)


def _mid1(aggx, xp, d0, d1, b, w1, w2, osplit):
    fin = xp.shape[1]
    fout = w2.shape[1]
    ins = aggx.shape[0]
    return pl.pallas_call(
        _mid1_body,
        grid=(NBLK,),
        in_specs=[
            _split_spec(ins, fin // ins),
            pl.BlockSpec((ROWS_BLK, fin), lambda i: (i, 0)),
            _col_spec(), _col_spec(),
            pl.BlockSpec((1, w1.shape[1]), lambda i: (0, 0)),
            pl.BlockSpec(w1.shape, lambda i: (0, 0)),
            pl.BlockSpec(w2.shape, lambda i: (0, 0)),
        ],
        out_specs=_split_spec(osplit, fout // osplit),
        out_shape=jax.ShapeDtypeStruct((osplit, NP, fout // osplit), jnp.float32),
    )(aggx, xp, d0, d1, b, w1, w2)


def _mid(agg, y, d0, d1, b, w, osplit):
    fin, fout = w.shape
    ins = agg.shape[0]
    return pl.pallas_call(
        _mid_body,
        grid=(NBLK,),
        in_specs=[
            _split_spec(ins, fin // ins),
            _split_spec(ins, fin // ins),
            _col_spec(), _col_spec(),
            pl.BlockSpec((1, fin), lambda i: (0, 0)),
            pl.BlockSpec((fin, fout), lambda i: (0, 0)),
        ],
        out_specs=_split_spec(osplit, fout // osplit),
        out_shape=jax.ShapeDtypeStruct((osplit, NP, fout // osplit), jnp.float32),
    )(agg, y, d0, d1, b, w)


def _fin(agg, y, d0, d1, b):
    fout = b.shape[1]
    ins = agg.shape[0]
    return pl.pallas_call(
        _fin_body,
        grid=(NBLK,),
        in_specs=[
            _split_spec(ins, fout // ins),
            _split_spec(ins, fout // ins),
            _col_spec(), _col_spec(),
            pl.BlockSpec((1, fout), lambda i: (0, 0)),
        ],
        out_specs=pl.BlockSpec((ROWS_BLK, fout), lambda i: (i, 0)),
        out_shape=jax.ShapeDtypeStruct((NP, fout), jnp.float32),
    )(agg, y, d0, d1, b)


def kernel(x, edge_index, W1, b1, W2, b2, W3, b3):
    src = edge_index[0].astype(jnp.int32)
    dst = edge_index[1].astype(jnp.int32)
    npad = EPAD - E
    fill = jnp.arange(npad, dtype=jnp.int32) % 8
    src_t = jnp.concatenate([src, fill]).reshape(TILES, CHUNKS, CHUNK)
    dst_t = jnp.concatenate([dst, N + fill]).reshape(TILES, CHUNKS, CHUNK)
    srcoff = jnp.stack([src_t, src_t + NP])

    xp = jnp.pad(x, ((0, NP - N), (0, 0)))

    deg = _DEG(dst_t)
    d0 = deg[0].reshape(NP, 1)
    d1 = deg[1].reshape(NP, 1)

    xd = _xd(xp, d0, d1, 2)
    aggx = _AGGX(xd.reshape(2 * NP, 64), srcoff, dst_t)
    y2 = _mid1(aggx, xp, d0, d1, b1.reshape(1, -1), W1, W2, 4)
    agg2 = _AGG64(y2.reshape(4 * NP, 64), srcoff, dst_t)
    y3 = _mid(agg2, y2, d0, d1, b2.reshape(1, -1), W3, 2)
    agg3 = _AGG32(y3.reshape(2 * NP, 32), srcoff, dst_t)
    out = _fin(agg3, y3, d0, d1, b3.reshape(1, -1))
    return out[:N]
